# Initial kernel scaffold; baseline (speedup 1.0000x reference)
import functools

import jax
import jax.numpy as jnp
from jax import lax
from jax.experimental import pallas as pl
from jax.experimental.pallas import tpu as pltpu
from jax.experimental.pallas import tpu_sc as plsc

N_NODES = 10000
DIM = 128
NPAD = 10240
ROWS_PER_TILE = NPAD // 16
NW = 32
CHUNK = 128

_MESH = plsc.VectorSubcoreMesh(core_axis_name="c", subcore_axis_name="s")


def _sc_degree(dst_r, ones16, zeros16, cpw):
    @functools.partial(
        pl.kernel,
        mesh=_MESH,
        out_type=jax.ShapeDtypeStruct((2, NPAD, 16), jnp.float32),
        scratch_types=[
            pltpu.VMEM((cpw, CHUNK), jnp.int32),
            pltpu.VMEM((CHUNK, 16), jnp.float32),
            pltpu.VMEM_SHARED((NPAD, 16), jnp.float32),
        ],
    )
    def deg_kernel(dst_hbm, ones_hbm, zeros_hbm, out_hbm, dst_v, ones_v, acc_s):
        c = lax.axis_index("c")
        s = lax.axis_index("s")
        wid = c * 16 + s
        rows = pl.ds(s * ROWS_PER_TILE, ROWS_PER_TILE)
        pltpu.sync_copy(zeros_hbm.at[rows], acc_s.at[rows])
        pltpu.sync_copy(ones_hbm, ones_v)
        pltpu.sync_copy(dst_hbm.at[wid], dst_v)
        plsc.subcore_barrier()

        @pl.loop(0, cpw)
        def _(j):
            pltpu.sync_copy(ones_v, acc_s.at[dst_v.at[j]], add=True)

        plsc.subcore_barrier()
        pltpu.sync_copy(acc_s.at[rows], out_hbm.at[c, rows])

    return deg_kernel(dst_r, ones16, zeros16)


def _sc_prop(hs, src_r, dst_r, zeros, cpw):
    @functools.partial(
        pl.kernel,
        mesh=_MESH,
        out_type=jax.ShapeDtypeStruct((2, NPAD, DIM), jnp.float32),
        scratch_types=[
            pltpu.VMEM((cpw, CHUNK), jnp.int32),
            pltpu.VMEM((cpw, CHUNK), jnp.int32),
            pltpu.VMEM((CHUNK, DIM), jnp.float32),
            pltpu.VMEM_SHARED((NPAD, DIM), jnp.float32),
        ],
    )
    def prop_kernel(hs_hbm, src_hbm, dst_hbm, zeros_hbm, out_hbm,
                    src_v, dst_v, row_v, acc_s):
        c = lax.axis_index("c")
        s = lax.axis_index("s")
        wid = c * 16 + s
        rows = pl.ds(s * ROWS_PER_TILE, ROWS_PER_TILE)

        @pl.when(c == 0)
        def _():
            pltpu.sync_copy(hs_hbm.at[rows], acc_s.at[rows])

        @pl.when(c == 1)
        def _():
            pltpu.sync_copy(zeros_hbm.at[rows], acc_s.at[rows])

        pltpu.sync_copy(src_hbm.at[wid], src_v)
        pltpu.sync_copy(dst_hbm.at[wid], dst_v)
        plsc.subcore_barrier()

        @pl.loop(0, cpw)
        def _(j):
            pltpu.sync_copy(hs_hbm.at[src_v.at[j]], row_v)
            pltpu.sync_copy(row_v, acc_s.at[dst_v.at[j]], add=True)

        plsc.subcore_barrier()
        pltpu.sync_copy(acc_s.at[rows], out_hbm.at[c, rows])

    return prop_kernel(hs, src_r, dst_r, zeros)


def _tc_prep(deg_parts, x_pad):
    rb = ROWS_PER_TILE

    def body(deg_ref, x_ref, dinv_ref, hs_ref):
        i = pl.program_id(0)
        deg = deg_ref[0, :, 0:1] + deg_ref[1, :, 0:1] + 1.0
        dinv = lax.rsqrt(deg)
        rowid = lax.broadcasted_iota(jnp.int32, (rb, 1), 0) + i * rb
        dinv = jnp.where(rowid < N_NODES, dinv, 0.0)
        dinvb = jnp.broadcast_to(dinv, (rb, DIM))
        dinv_ref[...] = dinvb
        hs_ref[...] = x_ref[...] * dinvb

    return pl.pallas_call(
        body,
        grid=(NPAD // rb,),
        in_specs=[
            pl.BlockSpec((2, rb, 16), lambda i: (0, i, 0)),
            pl.BlockSpec((rb, DIM), lambda i: (i, 0)),
        ],
        out_specs=[
            pl.BlockSpec((rb, DIM), lambda i: (i, 0)),
            pl.BlockSpec((rb, DIM), lambda i: (i, 0)),
        ],
        out_shape=[
            jax.ShapeDtypeStruct((NPAD, DIM), jnp.float32),
            jax.ShapeDtypeStruct((NPAD, DIM), jnp.float32),
        ],
    )(deg_parts, x_pad)


def _tc_layer(acc, dinvb, w, b2d, relu_scale):
    rb = ROWS_PER_TILE

    def body(acc_ref, dinv_ref, w_ref, b_ref, o_ref):
        dinv = dinv_ref[...]
        y = (acc_ref[0] + acc_ref[1]) * dinv
        h = jnp.dot(y, w_ref[...], preferred_element_type=jnp.float32,
                    precision=lax.Precision.HIGHEST) + b_ref[...]
        if relu_scale:
            o_ref[...] = jnp.maximum(h, 0.0) * dinv
        else:
            o_ref[...] = h

    return pl.pallas_call(
        body,
        grid=(NPAD // rb,),
        in_specs=[
            pl.BlockSpec((2, rb, DIM), lambda i: (0, i, 0)),
            pl.BlockSpec((rb, DIM), lambda i: (i, 0)),
            pl.BlockSpec((DIM, DIM), lambda i: (0, 0)),
            pl.BlockSpec((1, DIM), lambda i: (0, 0)),
        ],
        out_specs=pl.BlockSpec((rb, DIM), lambda i: (i, 0)),
        out_shape=jax.ShapeDtypeStruct((NPAD, DIM), jnp.float32),
    )(acc, dinvb, w, b2d)


def kernel(x, edge_index, W1, b1, W2, b2, W3, b3):
    n, d = x.shape
    e = edge_index.shape[1]
    cpw = -(-e // (NW * CHUNK))
    cpw += cpw % 2
    epad = NW * cpw * CHUNK

    src = edge_index[0].astype(jnp.int32)
    dst = edge_index[1].astype(jnp.int32)
    padv = jnp.full((epad - e,), n, dtype=jnp.int32)
    src_r = jnp.concatenate([src, padv]).reshape(NW, cpw, CHUNK)
    dst_r = jnp.concatenate([dst, padv]).reshape(NW, cpw, CHUNK)

    x_pad = jnp.pad(x.astype(jnp.float32), ((0, NPAD - n), (0, 0)))
    zeros = jnp.zeros((NPAD, DIM), jnp.float32)
    zeros16 = jnp.zeros((NPAD, 16), jnp.float32)
    ones16 = jnp.ones((CHUNK, 16), jnp.float32)

    deg_parts = _sc_degree(dst_r, ones16, zeros16, cpw)
    dinvb, hs = _tc_prep(deg_parts, x_pad)

    acc = _sc_prop(hs, src_r, dst_r, zeros, cpw)
    hs = _tc_layer(acc, dinvb, W1, b1.reshape(1, DIM), True)
    acc = _sc_prop(hs, src_r, dst_r, zeros, cpw)
    hs = _tc_layer(acc, dinvb, W2, b2.reshape(1, DIM), True)
    acc = _sc_prop(hs, src_r, dst_r, zeros, cpw)
    out = _tc_layer(acc, dinvb, W3, b3.reshape(1, DIM), False)
    return out[:n]


# speedup vs baseline: 5.9842x; 5.9842x over previous
import functools

import jax
import jax.numpy as jnp
from jax import lax
from jax.experimental import pallas as pl
from jax.experimental.pallas import tpu as pltpu
from jax.experimental.pallas import tpu_sc as plsc

N_NODES = 10000
DIM = 128
NPAD = 10240
ROWS_PER_TILE = NPAD // 16
NW = 32
CHUNK = 128

_MESH = plsc.VectorSubcoreMesh(core_axis_name="c", subcore_axis_name="s")


def _sc_prop(hs, src_r, dst_r, zeros, cpw):
    init2 = jnp.concatenate([hs, zeros], axis=0)

    @functools.partial(
        pl.kernel,
        mesh=_MESH,
        out_type=jax.ShapeDtypeStruct((2 * NPAD, DIM), jnp.float32),
        scratch_types=[
            pltpu.VMEM((cpw, CHUNK), jnp.int32),
            pltpu.VMEM((cpw, CHUNK), jnp.int32),
            pltpu.VMEM((CHUNK, DIM), jnp.float32),
            pltpu.VMEM_SHARED((NPAD, DIM), jnp.float32),
        ],
    )
    def prop_kernel(hs_hbm, src_hbm, dst_hbm, init_hbm, out_hbm,
                    src_v, dst_v, row_v, acc_s):
        c = lax.axis_index("c")
        s = lax.axis_index("s")
        wid = c * 16 + s
        rows = pl.ds(s * ROWS_PER_TILE, ROWS_PER_TILE)
        init_rows = pl.ds(c * NPAD + s * ROWS_PER_TILE, ROWS_PER_TILE)
        pltpu.sync_copy(init_hbm.at[init_rows], acc_s.at[rows])

        pltpu.sync_copy(src_hbm.at[wid], src_v)
        pltpu.sync_copy(dst_hbm.at[wid], dst_v)
        plsc.subcore_barrier()

        @pl.loop(0, cpw)
        def _(j):
            pltpu.sync_copy(hs_hbm.at[src_v.at[j]], row_v)
            pltpu.sync_copy(row_v, acc_s.at[dst_v.at[j]], add=True)

        plsc.subcore_barrier()
        pltpu.sync_copy(acc_s.at[rows], out_hbm.at[init_rows])

    return prop_kernel(hs, src_r, dst_r, init2).reshape(2, NPAD, DIM)


def _tc_prep(deg_acc, x_pad):
    rb = ROWS_PER_TILE

    def body(deg_ref, x_ref, dinv_ref, hs_ref):
        deg = deg_ref[0] + deg_ref[1]
        dinvb = jnp.where(deg >= 0.5, lax.rsqrt(jnp.maximum(deg, 0.5)), 0.0)
        dinv_ref[...] = dinvb
        hs_ref[...] = x_ref[...] * dinvb

    return pl.pallas_call(
        body,
        grid=(NPAD // rb,),
        in_specs=[
            pl.BlockSpec((2, rb, DIM), lambda i: (0, i, 0)),
            pl.BlockSpec((rb, DIM), lambda i: (i, 0)),
        ],
        out_specs=[
            pl.BlockSpec((rb, DIM), lambda i: (i, 0)),
            pl.BlockSpec((rb, DIM), lambda i: (i, 0)),
        ],
        out_shape=[
            jax.ShapeDtypeStruct((NPAD, DIM), jnp.float32),
            jax.ShapeDtypeStruct((NPAD, DIM), jnp.float32),
        ],
    )(deg_acc, x_pad)


def _tc_layer(acc, dinvb, w, b2d, relu_scale):
    rb = ROWS_PER_TILE

    def body(acc_ref, dinv_ref, w_ref, b_ref, o_ref):
        dinv = dinv_ref[...]
        y = (acc_ref[0] + acc_ref[1]) * dinv
        h = jnp.dot(y, w_ref[...], preferred_element_type=jnp.float32,
                    precision=lax.Precision.HIGHEST) + b_ref[...]
        if relu_scale:
            o_ref[...] = jnp.maximum(h, 0.0) * dinv
        else:
            o_ref[...] = h

    return pl.pallas_call(
        body,
        grid=(NPAD // rb,),
        in_specs=[
            pl.BlockSpec((2, rb, DIM), lambda i: (0, i, 0)),
            pl.BlockSpec((rb, DIM), lambda i: (i, 0)),
            pl.BlockSpec((DIM, DIM), lambda i: (0, 0)),
            pl.BlockSpec((1, DIM), lambda i: (0, 0)),
        ],
        out_specs=pl.BlockSpec((rb, DIM), lambda i: (i, 0)),
        out_shape=jax.ShapeDtypeStruct((NPAD, DIM), jnp.float32),
    )(acc, dinvb, w, b2d)


def kernel(x, edge_index, W1, b1, W2, b2, W3, b3):
    n, d = x.shape
    e = edge_index.shape[1]
    cpw = -(-e // (NW * CHUNK))
    cpw += cpw % 2
    epad = NW * cpw * CHUNK

    src = edge_index[0].astype(jnp.int32)
    dst = edge_index[1].astype(jnp.int32)
    padv = jnp.full((epad - e,), n, dtype=jnp.int32)
    src_r = jnp.concatenate([src, padv]).reshape(NW, cpw, CHUNK)
    dst_r = jnp.concatenate([dst, padv]).reshape(NW, cpw, CHUNK)

    x_pad = jnp.pad(x.astype(jnp.float32), ((0, NPAD - n), (0, 0)))
    zeros = jnp.zeros((NPAD, DIM), jnp.float32)
    ones_pad = jnp.pad(jnp.ones((n, DIM), jnp.float32), ((0, NPAD - n), (0, 0)))

    deg_acc = _sc_prop(ones_pad, src_r, dst_r, zeros, cpw)
    dinvb, hs = _tc_prep(deg_acc, x_pad)

    acc = _sc_prop(hs, src_r, dst_r, zeros, cpw)
    hs = _tc_layer(acc, dinvb, W1, b1.reshape(1, DIM), True)
    acc = _sc_prop(hs, src_r, dst_r, zeros, cpw)
    hs = _tc_layer(acc, dinvb, W2, b2.reshape(1, DIM), True)
    acc = _sc_prop(hs, src_r, dst_r, zeros, cpw)
    out = _tc_layer(acc, dinvb, W3, b3.reshape(1, DIM), False)
    return out[:n]


# speedup vs baseline: 8.0528x; 1.3457x over previous
import dataclasses
import functools

import jax
import jax.numpy as jnp
from jax import lax
from jax.experimental import pallas as pl
from jax.experimental.pallas import tpu as pltpu
from jax.experimental.pallas import tpu_sc as plsc

N_NODES = 10000
DIM = 128
NPAD = 10240
ROWS_PER_TILE = NPAD // 16
NW = 32
CHUNK = 128
NBUF = 2

_MESH = plsc.VectorSubcoreMesh(core_axis_name="c", subcore_axis_name="s")


def _sc_prop(hs, src_r, dst_r, zeros, cpw):
    init2 = jnp.concatenate([hs, zeros], axis=0)
    half = cpw // 2

    @functools.partial(
        pl.kernel,
        mesh=_MESH,
        out_type=jax.ShapeDtypeStruct((2 * NPAD, DIM), jnp.float32),
        scratch_types=[
            pltpu.VMEM((half, CHUNK), jnp.int32),
            pltpu.VMEM((half, CHUNK), jnp.int32),
            pltpu.VMEM_SHARED((NPAD, DIM), jnp.float32),
        ] + [pltpu.VMEM((CHUNK, DIM), jnp.float32)] * NBUF
          + [pltpu.SemaphoreType.DMA] * (2 * NBUF),
    )
    def prop_kernel(hs_hbm, src_hbm, dst_hbm, init_hbm, out_hbm,
                    src_v, dst_v, acc_s, *bufs_and_sems):
        bufs = bufs_and_sems[:NBUF]
        gsem = bufs_and_sems[NBUF:2 * NBUF]
        ssem = bufs_and_sems[2 * NBUF:]
        c = lax.axis_index("c")
        s = lax.axis_index("s")
        wid = c * 16 + s
        rows = pl.ds(s * ROWS_PER_TILE, ROWS_PER_TILE)
        io_rows = pl.ds(c * NPAD + s * ROWS_PER_TILE, ROWS_PER_TILE)

        def run_half(h, first):
            pltpu.sync_copy(src_hbm.at[wid, pl.ds(h * half, half)], src_v)
            pltpu.sync_copy(dst_hbm.at[wid, pl.ds(h * half, half)], dst_v)
            for b in range(NBUF):
                pltpu.async_copy(hs_hbm.at[src_v.at[b]], bufs[b], gsem[b])
            if first:
                pltpu.sync_copy(init_hbm.at[io_rows], acc_s.at[rows])
                plsc.subcore_barrier()

            @pl.loop(0, half - NBUF, step=NBUF)
            def _(j):
                for b in range(NBUF):
                    pltpu.make_async_copy(
                        hs_hbm.at[src_v.at[j + b]], bufs[b], gsem[b]).wait()
                    pltpu.async_copy(
                        bufs[b], acc_s.at[dst_v.at[j + b]], ssem[b], add=True)
                for b in range(NBUF):
                    pltpu.make_async_copy(
                        bufs[b], acc_s.at[dst_v.at[j + b]], ssem[b]).wait()
                    pltpu.async_copy(
                        hs_hbm.at[src_v.at[j + NBUF + b]], bufs[b], gsem[b])

            for b in range(NBUF):
                jb = half - NBUF + b
                pltpu.make_async_copy(
                    hs_hbm.at[src_v.at[jb]], bufs[b], gsem[b]).wait()
                pltpu.async_copy(
                    bufs[b], acc_s.at[dst_v.at[jb]], ssem[b], add=True)
            for b in range(NBUF):
                jb = half - NBUF + b
                pltpu.make_async_copy(
                    bufs[b], acc_s.at[dst_v.at[jb]], ssem[b]).wait()

        run_half(0, True)
        run_half(1, False)

        plsc.subcore_barrier()
        pltpu.sync_copy(acc_s.at[rows], out_hbm.at[io_rows])

    return prop_kernel(hs, src_r, dst_r, init2).reshape(2, NPAD, DIM)


def _sc_degree(dst_r, cpw):
    cp = pltpu.CompilerParams()
    if "needs_layout_passes" in pltpu.CompilerParams.__dataclass_fields__:
        cp = dataclasses.replace(cp, needs_layout_passes=False)

    @functools.partial(
        pl.kernel,
        mesh=_MESH,
        out_type=jax.ShapeDtypeStruct((NW, NPAD), jnp.float32),
        scratch_types=[
            pltpu.VMEM((cpw, CHUNK), jnp.int32),
            pltpu.VMEM((NPAD,), jnp.float32),
        ],
        compiler_params=cp,
    )
    def deg_kernel(dst_hbm, out_hbm, dst_v, histo):
        c = lax.axis_index("c")
        s = lax.axis_index("s")
        wid = c * 16 + s
        pltpu.sync_copy(dst_hbm.at[wid], dst_v)
        zero16 = jnp.zeros((16,), jnp.float32)
        ones16 = jnp.ones((16,), jnp.float32)

        @pl.loop(0, NPAD // 16)
        def _(i):
            histo[pl.ds(i * 16, 16)] = zero16

        @pl.loop(0, cpw)
        def _(j):
            @pl.loop(0, CHUNK // 16)
            def _(k):
                idx = dst_v[j, pl.ds(k * 16, 16)]
                plsc.addupdate_scatter(histo, [idx], ones16)

        pltpu.sync_copy(histo, out_hbm.at[wid])

    return deg_kernel(dst_r)


def _tc_prep(deg32, x_pad):
    rb = 2048
    qn = rb // DIM

    def body(deg_ref, x_ref, dinv_ref, hs_ref):
        i = pl.program_id(0)
        d = jnp.sum(deg_ref[...], axis=0) + 1.0
        node = (i * rb + lax.broadcasted_iota(jnp.int32, (qn, DIM), 0) * DIM
                + lax.broadcasted_iota(jnp.int32, (qn, DIM), 1))
        dinv = jnp.where(node < N_NODES, lax.rsqrt(d), 0.0)
        eye = jnp.where(
            lax.broadcasted_iota(jnp.int32, (DIM, DIM), 0)
            == lax.broadcasted_iota(jnp.int32, (DIM, DIM), 1), 1.0, 0.0)
        parts = []
        for q in range(qn):
            m = jnp.broadcast_to(dinv[q:q + 1, :], (DIM, DIM)) * eye
            col = jnp.sum(m, axis=1, keepdims=True)
            parts.append(jnp.broadcast_to(col, (DIM, DIM)))
        dinvb = jnp.concatenate(parts, axis=0)
        dinv_ref[...] = dinvb
        hs_ref[...] = x_ref[...] * dinvb

    return pl.pallas_call(
        body,
        grid=(NPAD // rb,),
        in_specs=[
            pl.BlockSpec((NW, qn, DIM), lambda i: (0, i, 0)),
            pl.BlockSpec((rb, DIM), lambda i: (i, 0)),
        ],
        out_specs=[
            pl.BlockSpec((rb, DIM), lambda i: (i, 0)),
            pl.BlockSpec((rb, DIM), lambda i: (i, 0)),
        ],
        out_shape=[
            jax.ShapeDtypeStruct((NPAD, DIM), jnp.float32),
            jax.ShapeDtypeStruct((NPAD, DIM), jnp.float32),
        ],
    )(deg32, x_pad)


def _tc_layer(acc, dinvb, w, b2d, relu_scale):
    rb = ROWS_PER_TILE

    def body(acc_ref, dinv_ref, w_ref, b_ref, o_ref):
        dinv = dinv_ref[...]
        y = (acc_ref[0] + acc_ref[1]) * dinv
        h = jnp.dot(y, w_ref[...], preferred_element_type=jnp.float32,
                    precision=lax.Precision.HIGHEST) + b_ref[...]
        if relu_scale:
            o_ref[...] = jnp.maximum(h, 0.0) * dinv
        else:
            o_ref[...] = h

    return pl.pallas_call(
        body,
        grid=(NPAD // rb,),
        in_specs=[
            pl.BlockSpec((2, rb, DIM), lambda i: (0, i, 0)),
            pl.BlockSpec((rb, DIM), lambda i: (i, 0)),
            pl.BlockSpec((DIM, DIM), lambda i: (0, 0)),
            pl.BlockSpec((1, DIM), lambda i: (0, 0)),
        ],
        out_specs=pl.BlockSpec((rb, DIM), lambda i: (i, 0)),
        out_shape=jax.ShapeDtypeStruct((NPAD, DIM), jnp.float32),
    )(acc, dinvb, w, b2d)


def kernel(x, edge_index, W1, b1, W2, b2, W3, b3):
    n, d = x.shape
    e = edge_index.shape[1]
    cpw = -(-e // (NW * CHUNK))
    cpw = -(-cpw // (2 * NBUF)) * 2 * NBUF
    epad = NW * cpw * CHUNK

    src = edge_index[0].astype(jnp.int32)
    dst = edge_index[1].astype(jnp.int32)
    padv = jnp.full((epad - e,), n, dtype=jnp.int32)
    src_r = jnp.concatenate([src, padv]).reshape(NW, cpw, CHUNK)
    dst_r = jnp.concatenate([dst, padv]).reshape(NW, cpw, CHUNK)

    x_pad = jnp.pad(x.astype(jnp.float32), ((0, NPAD - n), (0, 0)))
    zeros = jnp.zeros((NPAD, DIM), jnp.float32)

    deg32 = _sc_degree(dst_r, cpw).reshape(NW, NPAD // DIM, DIM)
    dinvb, hs = _tc_prep(deg32, x_pad)

    acc = _sc_prop(hs, src_r, dst_r, zeros, cpw)
    hs = _tc_layer(acc, dinvb, W1, b1.reshape(1, DIM), True)
    acc = _sc_prop(hs, src_r, dst_r, zeros, cpw)
    hs = _tc_layer(acc, dinvb, W2, b2.reshape(1, DIM), True)
    acc = _sc_prop(hs, src_r, dst_r, zeros, cpw)
    out = _tc_layer(acc, dinvb, W3, b3.reshape(1, DIM), False)
    return out[:n]


# speedup vs baseline: 8.9018x; 1.1054x over previous
import dataclasses
import functools

import jax
import jax.numpy as jnp
from jax import lax
from jax.experimental import pallas as pl
from jax.experimental.pallas import tpu as pltpu
from jax.experimental.pallas import tpu_sc as plsc

N_NODES = 10000
DIM = 128
NPAD = 10240
ROWS_PER_TILE = NPAD // 16
NW = 32
CHUNK = 128
NBUF = 2

_MESH = plsc.VectorSubcoreMesh(core_axis_name="c", subcore_axis_name="s")


def _sc_prop(hs, src_f, dst_f, zeros, cpw0, cpw1):
    init2 = jnp.concatenate([hs, zeros], axis=0)
    halfmax = max(cpw0, cpw1) // 2

    @functools.partial(
        pl.kernel,
        mesh=_MESH,
        out_type=jax.ShapeDtypeStruct((2 * NPAD, DIM), jnp.float32),
        scratch_types=[
            pltpu.VMEM((halfmax, CHUNK), jnp.int32),
            pltpu.VMEM((halfmax, CHUNK), jnp.int32),
            pltpu.VMEM_SHARED((NPAD, DIM), jnp.float32),
        ] + [pltpu.VMEM((CHUNK, DIM), jnp.float32)] * NBUF
          + [pltpu.SemaphoreType.DMA] * (2 * NBUF),
    )
    def prop_kernel(hs_hbm, src_hbm, dst_hbm, init_hbm, out_hbm,
                    src_v, dst_v, acc_s, *bufs_and_sems):
        bufs = bufs_and_sems[:NBUF]
        gsem = bufs_and_sems[NBUF:2 * NBUF]
        ssem = bufs_and_sems[2 * NBUF:]
        c = lax.axis_index("c")
        s = lax.axis_index("s")
        rows = pl.ds(s * ROWS_PER_TILE, ROWS_PER_TILE)
        io_rows = pl.ds(c * NPAD + s * ROWS_PER_TILE, ROWS_PER_TILE)

        def run_half(base, half, h):
            start = base + h * half
            pltpu.sync_copy(src_hbm.at[pl.ds(start, half)],
                            src_v.at[pl.ds(0, half)])
            pltpu.sync_copy(dst_hbm.at[pl.ds(start, half)],
                            dst_v.at[pl.ds(0, half)])
            for b in range(NBUF):
                pltpu.async_copy(hs_hbm.at[src_v.at[b]], bufs[b], gsem[b])

            @pl.loop(0, half - NBUF, step=NBUF)
            def _(j):
                for b in range(NBUF):
                    pltpu.make_async_copy(
                        hs_hbm.at[src_v.at[j + b]], bufs[b], gsem[b]).wait()
                    pltpu.async_copy(
                        bufs[b], acc_s.at[dst_v.at[j + b]], ssem[b], add=True)
                for b in range(NBUF):
                    pltpu.make_async_copy(
                        bufs[b], acc_s.at[dst_v.at[j + b]], ssem[b]).wait()
                    pltpu.async_copy(
                        hs_hbm.at[src_v.at[j + NBUF + b]], bufs[b], gsem[b])

            for b in range(NBUF):
                jb = half - NBUF + b
                pltpu.make_async_copy(
                    hs_hbm.at[src_v.at[jb]], bufs[b], gsem[b]).wait()
                pltpu.async_copy(
                    bufs[b], acc_s.at[dst_v.at[jb]], ssem[b], add=True)
            for b in range(NBUF):
                jb = half - NBUF + b
                pltpu.make_async_copy(
                    bufs[b], acc_s.at[dst_v.at[jb]], ssem[b]).wait()

        pltpu.sync_copy(init_hbm.at[io_rows], acc_s.at[rows])
        plsc.subcore_barrier()

        @pl.when(c == 0)
        def _():
            run_half(s * cpw0, cpw0 // 2, 0)
            run_half(s * cpw0, cpw0 // 2, 1)

        @pl.when(c == 1)
        def _():
            base = 16 * cpw0 + s * cpw1
            run_half(base, cpw1 // 2, 0)
            run_half(base, cpw1 // 2, 1)

        plsc.subcore_barrier()
        pltpu.sync_copy(acc_s.at[rows], out_hbm.at[io_rows])

    return prop_kernel(hs, src_f, dst_f, init2).reshape(2, NPAD, DIM)


def _sc_degree(dst_r, cpw):
    cp = pltpu.CompilerParams()
    if "needs_layout_passes" in pltpu.CompilerParams.__dataclass_fields__:
        cp = dataclasses.replace(cp, needs_layout_passes=False)

    @functools.partial(
        pl.kernel,
        mesh=_MESH,
        out_type=jax.ShapeDtypeStruct((NW, NPAD), jnp.float32),
        scratch_types=[
            pltpu.VMEM((cpw, CHUNK), jnp.int32),
            pltpu.VMEM((NPAD,), jnp.float32),
        ],
        compiler_params=cp,
    )
    def deg_kernel(dst_hbm, out_hbm, dst_v, histo):
        c = lax.axis_index("c")
        s = lax.axis_index("s")
        wid = c * 16 + s
        pltpu.sync_copy(dst_hbm.at[wid], dst_v)
        zero16 = jnp.zeros((16,), jnp.float32)
        ones16 = jnp.ones((16,), jnp.float32)

        @pl.loop(0, NPAD // 16)
        def _(i):
            histo[pl.ds(i * 16, 16)] = zero16

        @pl.loop(0, cpw)
        def _(j):
            @pl.loop(0, CHUNK // 16)
            def _(k):
                idx = dst_v[j, pl.ds(k * 16, 16)]
                plsc.addupdate_scatter(histo, [idx], ones16)

        pltpu.sync_copy(histo, out_hbm.at[wid])

    return deg_kernel(dst_r)


def _tc_prep(deg32, x_pad):
    rb = 2048
    qn = rb // DIM

    def body(deg_ref, x_ref, dinv_ref, hs_ref):
        i = pl.program_id(0)
        d = jnp.sum(deg_ref[...], axis=0) + 1.0
        node = (i * rb + lax.broadcasted_iota(jnp.int32, (qn, DIM), 0) * DIM
                + lax.broadcasted_iota(jnp.int32, (qn, DIM), 1))
        dinv = jnp.where(node < N_NODES, lax.rsqrt(d), 0.0)
        eye = jnp.where(
            lax.broadcasted_iota(jnp.int32, (DIM, DIM), 0)
            == lax.broadcasted_iota(jnp.int32, (DIM, DIM), 1), 1.0, 0.0)
        parts = []
        for q in range(qn):
            m = jnp.broadcast_to(dinv[q:q + 1, :], (DIM, DIM)) * eye
            col = jnp.sum(m, axis=1, keepdims=True)
            parts.append(jnp.broadcast_to(col, (DIM, DIM)))
        dinvb = jnp.concatenate(parts, axis=0)
        dinv_ref[...] = dinvb
        hs_ref[...] = x_ref[...] * dinvb

    return pl.pallas_call(
        body,
        grid=(NPAD // rb,),
        in_specs=[
            pl.BlockSpec((NW, qn, DIM), lambda i: (0, i, 0)),
            pl.BlockSpec((rb, DIM), lambda i: (i, 0)),
        ],
        out_specs=[
            pl.BlockSpec((rb, DIM), lambda i: (i, 0)),
            pl.BlockSpec((rb, DIM), lambda i: (i, 0)),
        ],
        out_shape=[
            jax.ShapeDtypeStruct((NPAD, DIM), jnp.float32),
            jax.ShapeDtypeStruct((NPAD, DIM), jnp.float32),
        ],
    )(deg32, x_pad)


def _tc_layer(acc, dinvb, w, b2d, relu_scale):
    rb = ROWS_PER_TILE

    def body(acc_ref, dinv_ref, w_ref, b_ref, o_ref):
        dinv = dinv_ref[...]
        y = (acc_ref[0] + acc_ref[1]) * dinv
        h = jnp.dot(y, w_ref[...], preferred_element_type=jnp.float32,
                    precision=lax.Precision.HIGHEST) + b_ref[...]
        if relu_scale:
            o_ref[...] = jnp.maximum(h, 0.0) * dinv
        else:
            o_ref[...] = h

    return pl.pallas_call(
        body,
        grid=(NPAD // rb,),
        in_specs=[
            pl.BlockSpec((2, rb, DIM), lambda i: (0, i, 0)),
            pl.BlockSpec((rb, DIM), lambda i: (i, 0)),
            pl.BlockSpec((DIM, DIM), lambda i: (0, 0)),
            pl.BlockSpec((1, DIM), lambda i: (0, 0)),
        ],
        out_specs=pl.BlockSpec((rb, DIM), lambda i: (i, 0)),
        out_shape=jax.ShapeDtypeStruct((NPAD, DIM), jnp.float32),
    )(acc, dinvb, w, b2d)


def kernel(x, edge_index, W1, b1, W2, b2, W3, b3):
    n, d = x.shape
    e = edge_index.shape[1]
    cpw = -(-e // (NW * CHUNK))
    cpw = -(-cpw // (2 * NBUF)) * 2 * NBUF
    epad = NW * cpw * CHUNK
    cpw0 = int(round(2 * cpw * 0.79 / 8)) * 8
    cpw1 = 2 * cpw - cpw0

    src = edge_index[0].astype(jnp.int32)
    dst = edge_index[1].astype(jnp.int32)
    padv = jnp.full((epad - e,), n, dtype=jnp.int32)
    src_f = jnp.concatenate([src, padv]).reshape(epad // CHUNK, CHUNK)
    dst_f = jnp.concatenate([dst, padv]).reshape(epad // CHUNK, CHUNK)
    dst_r = dst_f.reshape(NW, cpw, CHUNK)

    x_pad = jnp.pad(x.astype(jnp.float32), ((0, NPAD - n), (0, 0)))
    zeros = jnp.zeros((NPAD, DIM), jnp.float32)

    deg32 = _sc_degree(dst_r, cpw).reshape(NW, NPAD // DIM, DIM)
    dinvb, hs = _tc_prep(deg32, x_pad)

    acc = _sc_prop(hs, src_f, dst_f, zeros, cpw0, cpw1)
    hs = _tc_layer(acc, dinvb, W1, b1.reshape(1, DIM), True)
    acc = _sc_prop(hs, src_f, dst_f, zeros, cpw0, cpw1)
    hs = _tc_layer(acc, dinvb, W2, b2.reshape(1, DIM), True)
    acc = _sc_prop(hs, src_f, dst_f, zeros, cpw0, cpw1)
    out = _tc_layer(acc, dinvb, W3, b3.reshape(1, DIM), False)
    return out[:n]
